# Initial kernel scaffold; baseline (speedup 1.0000x reference)
import jax
import jax.numpy as jnp
from jax.experimental import pallas as pl


def kernel(labels, table):
    raise NotImplementedError("write your pallas kernel here")


# speedup vs baseline: 1.4924x; 1.4924x over previous
import functools

import jax
import jax.numpy as jnp
from jax import lax
from jax.experimental import pallas as pl
from jax.experimental.pallas import tpu as pltpu
from jax.experimental.pallas import tpu_sc as plsc

_B = 16384
_D = 1024
_V = 1000

_info = plsc.get_sparse_core_info()
_NC = _info.num_cores
_NS = _info.num_subcores
_NW = _NC * _NS
_BPW = _B // _NW
_C = 64
_NCHUNK = _BPW // _C

_mesh = plsc.VectorSubcoreMesh(core_axis_name="c", subcore_axis_name="s")


@functools.partial(
    pl.kernel,
    mesh=_mesh,
    out_type=jax.ShapeDtypeStruct((_B, _D), jnp.float32),
    scratch_types=[
        pltpu.VMEM((_C,), jnp.int32),
        pltpu.VMEM((_C, _D), jnp.float32),
        pltpu.SemaphoreType.DMA,
    ],
)
def _embed_sc(labels_hbm, table_hbm, out_hbm, idx_v, rows_v, sem):
    wid = lax.axis_index("s") * _NC + lax.axis_index("c")
    base = wid * _BPW
    for g in range(_NCHUNK):
        off = base + g * _C
        pltpu.sync_copy(labels_hbm.at[pl.ds(off, _C)], idx_v)
        pltpu.async_copy(table_hbm.at[idx_v], rows_v, sem).wait()
        pltpu.sync_copy(rows_v, out_hbm.at[pl.ds(off, _C)])


def kernel(labels, table):
    return _embed_sc(labels.astype(jnp.int32), table)


# speedup vs baseline: 1.5904x; 1.0657x over previous
import functools

import jax
import jax.numpy as jnp
from jax import lax
from jax.experimental import pallas as pl
from jax.experimental.pallas import tpu as pltpu
from jax.experimental.pallas import tpu_sc as plsc

_B = 16384
_D = 1024
_V = 1000

_info = plsc.get_sparse_core_info()
_NC = _info.num_cores
_NS = _info.num_subcores
_NW = _NC * _NS
_BPW = _B // _NW
_C = 32
_NCHUNK = _BPW // _C

_mesh = plsc.VectorSubcoreMesh(core_axis_name="c", subcore_axis_name="s")


@functools.partial(
    pl.kernel,
    mesh=_mesh,
    out_type=jax.ShapeDtypeStruct((_B, _D), jnp.float32),
    scratch_types=[
        pltpu.VMEM((_C,), jnp.int32),
        pltpu.VMEM((_C,), jnp.int32),
        pltpu.VMEM((_C, _D), jnp.float32),
        pltpu.VMEM((_C, _D), jnp.float32),
        pltpu.SemaphoreType.DMA,
        pltpu.SemaphoreType.DMA,
    ],
)
def _embed_sc(labels_hbm, table_hbm, out_hbm, idx0, idx1, rows0, rows1,
              sem0, sem1):
    wid = lax.axis_index("s") * _NC + lax.axis_index("c")
    base = wid * _BPW
    idx = (idx0, idx1)
    rows = (rows0, rows1)
    sem = (sem0, sem1)
    gathers = [None, None]
    pltpu.sync_copy(labels_hbm.at[pl.ds(base, _C)], idx[0])
    gathers[0] = pltpu.async_copy(table_hbm.at[idx[0]], rows[0], sem[0])
    for g in range(1, _NCHUNK):
        b = g % 2
        p = 1 - b
        pltpu.sync_copy(labels_hbm.at[pl.ds(base + g * _C, _C)], idx[b])
        gathers[b] = pltpu.async_copy(table_hbm.at[idx[b]], rows[b], sem[b])
        gathers[p].wait()
        pltpu.sync_copy(rows[p], out_hbm.at[pl.ds(base + (g - 1) * _C, _C)])
    last = (_NCHUNK - 1) % 2
    gathers[last].wait()
    pltpu.sync_copy(rows[last], out_hbm.at[pl.ds(base + (_NCHUNK - 1) * _C, _C)])


def kernel(labels, table):
    return _embed_sc(labels.astype(jnp.int32), table)
